# baseline (speedup 1.0000x reference)
import functools

import jax
import jax.numpy as jnp
from jax import lax
from jax.experimental import pallas as pl
from jax.experimental.pallas import tpu as pltpu
from jax.experimental.pallas import tpu_sc as plsc

_MB = 256
_KC = 2048


def _argmin_body(x_ref, c_ref, w_in_ref, b_in_ref, idx_ref):
    K = c_ref.shape[0]
    xb = x_ref[0]
    p = jnp.dot(w_in_ref[...], xb, preferred_element_type=jnp.float32)
    p = p + b_in_ref[...]

    cnorm = jnp.sum(c_ref[...] * c_ref[...], axis=1, keepdims=True)

    run_min = jnp.full((1, _MB), jnp.inf, dtype=jnp.float32)
    run_idx = jnp.zeros((1, _MB), dtype=jnp.int32)
    for k0 in range(0, K, _KC):
        d = cnorm[k0:k0 + _KC, :] - 2.0 * jnp.dot(
            c_ref[pl.ds(k0, _KC), :], p,
            preferred_element_type=jnp.float32)
        mnc = jnp.min(d, axis=0, keepdims=True)
        iota = lax.broadcasted_iota(jnp.int32, (_KC, _MB), 0) + k0
        idxc = jnp.min(jnp.where(d == mnc, iota, K), axis=0, keepdims=True)
        better = mnc < run_min
        run_idx = jnp.where(better, idxc, run_idx)
        run_min = jnp.minimum(run_min, mnc)
    idx_ref[0, 0] = run_idx[0]


def _proj_out_body(rows_ref, idx_ref, w_out_ref, b_out_ref, out_ref,
                   near_ref):
    L = near_ref.shape[2]
    rows = rows_ref[0]
    sub = idx_ref[0, 0][:, None] & 3
    nb = jnp.zeros((rows.shape[0], L), dtype=jnp.float32)
    for q in range(4):
        nb = nb + jnp.where(sub == q, rows[:, q * L:(q + 1) * L], 0.0)
    near_ref[0] = nb
    out_t = lax.dot_general(w_out_ref[...], nb,
                            (((1,), (1,)), ((), ())),
                            preferred_element_type=jnp.float32)
    out_ref[0] = out_t + b_out_ref[...]


def _make_sc_gather(Vq, D, B):
    info = plsc.get_sparse_core_info()
    nw = info.num_cores * info.num_subcores
    b_per_w = B // nw
    mesh = plsc.VectorSubcoreMesh(core_axis_name="c", subcore_axis_name="s")

    @functools.partial(
        pl.kernel, mesh=mesh,
        out_type=jax.ShapeDtypeStruct((B, D), jnp.float32),
        scratch_types=[
            pltpu.VMEM((b_per_w,), jnp.int32),
            pltpu.VMEM((b_per_w,), jnp.int32),
            pltpu.VMEM((b_per_w, D), jnp.float32),
            pltpu.SemaphoreType.DMA,
        ],
    )
    def gather(table_hbm, idx_hbm, out_hbm, idx_v, idxq_v, rows_v, sem):
        wid = lax.axis_index("s") * info.num_cores + lax.axis_index("c")
        base = wid * b_per_w
        pltpu.sync_copy(idx_hbm.at[pl.ds(base, b_per_w)], idx_v)
        idxq_v[...] = idx_v[...] >> 2
        pltpu.async_copy(table_hbm.at[idxq_v], rows_v, sem).wait()
        pltpu.sync_copy(rows_v, out_hbm.at[pl.ds(base, b_per_w)])

    return gather


def kernel(x, codebook, W_in, b_in, W_out, b_out):
    b, c, h, w = x.shape
    hw = h * w
    K, L = codebook.shape[2], codebook.shape[3]
    xf = x.reshape(b, c, hw)
    C = codebook.reshape(K, L)
    n_blocks = b * (hw // _MB)

    grid = (b, hw // _MB)
    idx3 = pl.pallas_call(
        _argmin_body,
        grid=grid,
        in_specs=[
            pl.BlockSpec((1, c, _MB), lambda i, j: (i, 0, j)),
            pl.BlockSpec((K, L), lambda i, j: (0, 0)),
            pl.BlockSpec((L, c), lambda i, j: (0, 0)),
            pl.BlockSpec((L, 1), lambda i, j: (0, 0)),
        ],
        out_specs=pl.BlockSpec((1, 1, _MB),
                               lambda i, j: (i * (hw // _MB) + j, 0, 0)),
        out_shape=jax.ShapeDtypeStruct((n_blocks, 1, _MB), jnp.int32),
        compiler_params=pltpu.CompilerParams(
            dimension_semantics=("parallel", "parallel")),
    )(xf, C, W_in, b_in.reshape(L, 1))

    idx_flat = idx3.reshape(b * hw)
    DP = 128
    Cq = C.reshape(K * L // DP, DP)
    rows_flat = _make_sc_gather(Cq.shape[0], DP, b * hw)(Cq, idx_flat)
    rows = rows_flat.reshape(b, hw, DP)

    out_t, nearest = pl.pallas_call(
        _proj_out_body,
        grid=grid,
        in_specs=[
            pl.BlockSpec((1, _MB, DP), lambda i, j: (i, j, 0)),
            pl.BlockSpec((1, 1, _MB),
                         lambda i, j: (i * (hw // _MB) + j, 0, 0)),
            pl.BlockSpec((c, L), lambda i, j: (0, 0)),
            pl.BlockSpec((c, 1), lambda i, j: (0, 0)),
        ],
        out_specs=[
            pl.BlockSpec((1, c, _MB), lambda i, j: (i, 0, j)),
            pl.BlockSpec((1, _MB, L), lambda i, j: (i, j, 0)),
        ],
        out_shape=[
            jax.ShapeDtypeStruct((b, c, hw), jnp.float32),
            jax.ShapeDtypeStruct((b, hw, L), jnp.float32),
        ],
        compiler_params=pltpu.CompilerParams(
            dimension_semantics=("parallel", "parallel")),
    )(rows, idx3, W_out, b_out.reshape(c, 1))

    return out_t.reshape(b, c, h, w), nearest


# speedup vs baseline: 1.1670x; 1.1670x over previous
import functools

import jax
import jax.numpy as jnp
from jax import lax
from jax.experimental import pallas as pl
from jax.experimental.pallas import tpu as pltpu
from jax.experimental.pallas import tpu_sc as plsc

_MB = 1024
_KC = 2048


def _argmin_body(x_ref, c_ref, w_in_ref, b_in_ref, idx_ref):
    K = c_ref.shape[0]
    xb = x_ref[0]
    p = jnp.dot(w_in_ref[...], xb, preferred_element_type=jnp.float32)
    p = p + b_in_ref[...]

    cnorm = jnp.sum(c_ref[...] * c_ref[...], axis=1, keepdims=True)

    run_min = jnp.full((1, _MB), jnp.inf, dtype=jnp.float32)
    run_idx = jnp.zeros((1, _MB), dtype=jnp.int32)
    for k0 in range(0, K, _KC):
        d = cnorm[k0:k0 + _KC, :] - 2.0 * jnp.dot(
            c_ref[pl.ds(k0, _KC), :], p,
            preferred_element_type=jnp.float32)
        mnc = jnp.min(d, axis=0, keepdims=True)
        iota = lax.broadcasted_iota(jnp.int32, (_KC, _MB), 0) + k0
        idxc = jnp.min(jnp.where(d == mnc, iota, K), axis=0, keepdims=True)
        better = mnc < run_min
        run_idx = jnp.where(better, idxc, run_idx)
        run_min = jnp.minimum(run_min, mnc)
    idx_ref[0, 0] = run_idx[0]


def _proj_out_body(rows_ref, idx_ref, w_out_ref, b_out_ref, out_ref,
                   near_ref):
    L = near_ref.shape[2]
    rows = rows_ref[0]
    sub = idx_ref[0, 0][:, None] & 3
    nb = jnp.zeros((rows.shape[0], L), dtype=jnp.float32)
    for q in range(4):
        nb = nb + jnp.where(sub == q, rows[:, q * L:(q + 1) * L], 0.0)
    near_ref[0] = nb
    out_t = lax.dot_general(w_out_ref[...], nb,
                            (((1,), (1,)), ((), ())),
                            preferred_element_type=jnp.float32)
    out_ref[0] = out_t + b_out_ref[...]


def _make_sc_gather(Vq, D, B):
    info = plsc.get_sparse_core_info()
    nw = info.num_subcores
    b_per_w = B // nw
    mesh = plsc.VectorSubcoreMesh(core_axis_name="c", subcore_axis_name="s",
                                  num_cores=1)

    @functools.partial(
        pl.kernel, mesh=mesh,
        out_type=jax.ShapeDtypeStruct((B, D), jnp.float32),
        scratch_types=[
            pltpu.VMEM((b_per_w,), jnp.int32),
            pltpu.VMEM((b_per_w,), jnp.int32),
            pltpu.VMEM((b_per_w, D), jnp.float32),
            pltpu.SemaphoreType.DMA,
        ],
    )
    def gather(table_hbm, idx_hbm, out_hbm, idx_v, idxq_v, rows_v, sem):
        wid = lax.axis_index("s")
        base = wid * b_per_w
        pltpu.sync_copy(idx_hbm.at[pl.ds(base, b_per_w)], idx_v)
        idxq_v[...] = idx_v[...] >> 2
        pltpu.async_copy(table_hbm.at[idxq_v], rows_v, sem).wait()
        pltpu.sync_copy(rows_v, out_hbm.at[pl.ds(base, b_per_w)])

    return gather


def kernel(x, codebook, W_in, b_in, W_out, b_out):
    b, c, h, w = x.shape
    hw = h * w
    K, L = codebook.shape[2], codebook.shape[3]
    xf = x.reshape(b, c, hw)
    C = codebook.reshape(K, L)
    n_blocks = b * (hw // _MB)

    grid = (b, hw // _MB)
    idx3 = pl.pallas_call(
        _argmin_body,
        grid=grid,
        in_specs=[
            pl.BlockSpec((1, c, _MB), lambda i, j: (i, 0, j)),
            pl.BlockSpec((K, L), lambda i, j: (0, 0)),
            pl.BlockSpec((L, c), lambda i, j: (0, 0)),
            pl.BlockSpec((L, 1), lambda i, j: (0, 0)),
        ],
        out_specs=pl.BlockSpec((1, 1, _MB),
                               lambda i, j: (i * (hw // _MB) + j, 0, 0)),
        out_shape=jax.ShapeDtypeStruct((n_blocks, 1, _MB), jnp.int32),
        compiler_params=pltpu.CompilerParams(
            dimension_semantics=("parallel", "parallel")),
    )(xf, C, W_in, b_in.reshape(L, 1))

    idx_flat = idx3.reshape(b * hw)
    DP = 128
    Cq = C.reshape(K * L // DP, DP)
    rows_flat = _make_sc_gather(Cq.shape[0], DP, b * hw)(Cq, idx_flat)
    rows = rows_flat.reshape(b, hw, DP)

    out_t, nearest = pl.pallas_call(
        _proj_out_body,
        grid=grid,
        in_specs=[
            pl.BlockSpec((1, _MB, DP), lambda i, j: (i, j, 0)),
            pl.BlockSpec((1, 1, _MB),
                         lambda i, j: (i * (hw // _MB) + j, 0, 0)),
            pl.BlockSpec((c, L), lambda i, j: (0, 0)),
            pl.BlockSpec((c, 1), lambda i, j: (0, 0)),
        ],
        out_specs=[
            pl.BlockSpec((1, c, _MB), lambda i, j: (i, 0, j)),
            pl.BlockSpec((1, _MB, L), lambda i, j: (i, j, 0)),
        ],
        out_shape=[
            jax.ShapeDtypeStruct((b, c, hw), jnp.float32),
            jax.ShapeDtypeStruct((b, hw, L), jnp.float32),
        ],
        compiler_params=pltpu.CompilerParams(
            dimension_semantics=("parallel", "parallel")),
    )(rows, idx3, W_out, b_out.reshape(c, 1))

    return out_t.reshape(b, c, h, w), nearest
